# baseline (speedup 1.0000x reference)
import functools

import jax
import jax.numpy as jnp
from jax.experimental import pallas as pl
from jax.experimental.pallas import tpu as pltpu

_B = 4096
_BLK = 512


def _loss_block_kernel(logits_ref, bias_ref, out_ref):
    i = pl.program_id(0)
    x = logits_ref[...] + bias_ref[0]
    rows = jax.lax.broadcasted_iota(jnp.int32, x.shape, 0) + i * _BLK
    cols = jax.lax.broadcasted_iota(jnp.int32, x.shape, 1)
    signed = jnp.where(rows == cols, x, -x)
    s = jnp.sum(jnp.minimum(signed, 0.0) - jnp.log1p(jnp.exp(-jnp.abs(signed))))

    @pl.when(i == 0)
    def _init():
        out_ref[0] = 0.0

    out_ref[0] += s


@jax.jit
def kernel(logits, text_emb, image_emb, logit_bias):
    B = logits.shape[0]
    bias = jnp.reshape(logit_bias, (1,)).astype(jnp.float32)
    total = pl.pallas_call(
        _loss_block_kernel,
        grid=(B // _BLK,),
        in_specs=[
            pl.BlockSpec((_BLK, B), lambda i: (i, 0)),
            pl.BlockSpec(memory_space=pltpu.SMEM),
        ],
        out_specs=pl.BlockSpec(memory_space=pltpu.SMEM),
        out_shape=jax.ShapeDtypeStruct((1,), jnp.float32),
    )(logits, bias)
    return -total[0] / (B * B)
